# Initial kernel scaffold; baseline (speedup 1.0000x reference)
import jax
import jax.numpy as jnp
from jax.experimental import pallas as pl


def kernel(x, adj, W1, b1, W2, b2):
    raise NotImplementedError("write your pallas kernel here")


# speedup vs baseline: 1.0106x; 1.0106x over previous
import jax
import jax.numpy as jnp
from jax.experimental import pallas as pl
from jax.experimental.pallas import tpu as pltpu

_BM = 400


def _body(adj_ref, x_ref, w1t_ref, b1_ref, w2t_ref, b2_ref,
          y_ref, emb_ref, h_ref):
    p = pl.program_id(0)
    i = pl.program_id(1)
    ab = adj_ref[...].astype(jnp.bfloat16)

    @pl.when(p == 0)
    def _phase0():
        pre = jnp.dot(ab, x_ref[...], preferred_element_type=jnp.float32)
        hblk = jax.lax.dot_general(
            pre.astype(jnp.bfloat16), w1t_ref[...], (((1,), (0,)), ((), ())),
            preferred_element_type=jnp.float32) + b1_ref[...]
        h_ref[pl.ds(i * _BM, _BM), :] = jnp.maximum(hblk, 0.0).astype(jnp.bfloat16)

    @pl.when(p == 1)
    def _phase1():
        emb = jnp.dot(ab, h_ref[...], preferred_element_type=jnp.float32)
        emb_ref[...] = emb
        y_ref[...] = jax.lax.dot_general(
            emb.astype(jnp.bfloat16), w2t_ref[...], (((1,), (0,)), ((), ())),
            preferred_element_type=jnp.float32) + b2_ref[...]


def kernel(x, adj, W1, b1, W2, b2):
    n, xd = x.shape
    hd = W1.shape[0]
    yd = W2.shape[0]
    nb = n // _BM

    y, emb = pl.pallas_call(
        _body,
        grid=(2, nb),
        in_specs=[
            pl.BlockSpec((_BM, n), lambda p, i: (i, 0)),
            pl.BlockSpec((n, xd), lambda p, i: (0, 0)),
            pl.BlockSpec((xd, hd), lambda p, i: (0, 0)),
            pl.BlockSpec((1, hd), lambda p, i: (0, 0)),
            pl.BlockSpec((hd, yd), lambda p, i: (0, 0)),
            pl.BlockSpec((1, yd), lambda p, i: (0, 0)),
        ],
        out_specs=[
            pl.BlockSpec((_BM, yd), lambda p, i: (p * i, 0)),
            pl.BlockSpec((_BM, hd), lambda p, i: (p * i, 0)),
        ],
        out_shape=[
            jax.ShapeDtypeStruct((n, yd), jnp.float32),
            jax.ShapeDtypeStruct((n, hd), jnp.float32),
        ],
        scratch_shapes=[pltpu.VMEM((n, hd), jnp.bfloat16)],
        compiler_params=pltpu.CompilerParams(
            dimension_semantics=("arbitrary", "arbitrary"),
        ),
        interpret=False,
    )(adj, x.astype(jnp.bfloat16), W1.T.astype(jnp.bfloat16),
      b1.reshape(1, hd), W2.T.astype(jnp.bfloat16), b2.reshape(1, yd))
    return (y, emb)


# speedup vs baseline: 1.0336x; 1.0228x over previous
import jax
import jax.numpy as jnp
from jax.experimental import pallas as pl
from jax.experimental.pallas import tpu as pltpu

_BM = 400


def _body(adj_ref, x_ref, w1_ref, b1_ref, w2_ref, b2_ref,
          y_ref, emb_ref, xb_ref, h_ref):
    p = pl.program_id(0)
    i = pl.program_id(1)
    ab = adj_ref[...].astype(jnp.bfloat16)

    @pl.when((p == 0) & (i == 0))
    def _cast_x():
        xb_ref[...] = x_ref[...].astype(jnp.bfloat16)

    @pl.when(p == 0)
    def _phase0():
        pre = jnp.dot(ab, xb_ref[...], preferred_element_type=jnp.float32)
        hblk = jax.lax.dot_general(
            pre.astype(jnp.bfloat16), w1_ref[...].astype(jnp.bfloat16),
            (((1,), (1,)), ((), ())),
            preferred_element_type=jnp.float32) + b1_ref[...]
        h_ref[pl.ds(i * _BM, _BM), :] = jnp.maximum(hblk, 0.0).astype(jnp.bfloat16)

    @pl.when(p == 1)
    def _phase1():
        emb = jnp.dot(ab, h_ref[...], preferred_element_type=jnp.float32)
        emb_ref[...] = emb
        y_ref[...] = jax.lax.dot_general(
            emb.astype(jnp.bfloat16), w2_ref[...].astype(jnp.bfloat16),
            (((1,), (1,)), ((), ())),
            preferred_element_type=jnp.float32) + b2_ref[...]


def kernel(x, adj, W1, b1, W2, b2):
    n, xd = x.shape
    hd = W1.shape[0]
    yd = W2.shape[0]
    nb = n // _BM

    y, emb = pl.pallas_call(
        _body,
        grid=(2, nb),
        in_specs=[
            pl.BlockSpec((_BM, n), lambda p, i: (i, 0)),
            pl.BlockSpec((n, xd), lambda p, i: (0, 0)),
            pl.BlockSpec((hd, xd), lambda p, i: (0, 0)),
            pl.BlockSpec((1, hd), lambda p, i: (0, 0)),
            pl.BlockSpec((yd, hd), lambda p, i: (0, 0)),
            pl.BlockSpec((1, yd), lambda p, i: (0, 0)),
        ],
        out_specs=[
            pl.BlockSpec((_BM, yd), lambda p, i: (p * i, 0)),
            pl.BlockSpec((_BM, hd), lambda p, i: (p * i, 0)),
        ],
        out_shape=[
            jax.ShapeDtypeStruct((n, yd), jnp.float32),
            jax.ShapeDtypeStruct((n, hd), jnp.float32),
        ],
        scratch_shapes=[
            pltpu.VMEM((n, xd), jnp.bfloat16),
            pltpu.VMEM((n, hd), jnp.bfloat16),
        ],
        compiler_params=pltpu.CompilerParams(
            dimension_semantics=("arbitrary", "arbitrary"),
        ),
        interpret=False,
    )(adj, x, W1, b1.reshape(1, hd), W2, b2.reshape(1, yd))
    return (y, emb)
